# baseline (speedup 1.0000x reference)
import functools

import jax
import jax.numpy as jnp
from jax import lax
from jax.experimental import pallas as pl
from jax.experimental.pallas import tpu as pltpu
from jax.experimental.pallas import tpu_sc as plsc

_GCHUNK = 64


def _sc_gather(table, ids4):
    num_workers, chunks_per_w, _, chunk = ids4.shape
    depth = table.shape[1]
    out_rows = num_workers * chunks_per_w * chunk

    info = plsc.get_sparse_core_info()
    num_cores = info.num_cores

    mesh = plsc.VectorSubcoreMesh(core_axis_name="c", subcore_axis_name="s")

    @functools.partial(
        pl.kernel,
        mesh=mesh,
        out_type=jax.ShapeDtypeStruct((out_rows, 2 * depth), jnp.float32),
        scratch_types=[
            pltpu.VMEM((chunks_per_w, 2, chunk), jnp.int32),
            pltpu.VMEM((2, chunk, depth), jnp.float32),
            pltpu.SemaphoreType.DMA,
        ],
        compiler_params=pltpu.CompilerParams(use_tc_tiling_on_sc=False),
    )
    def gather_kernel(table_hbm, idx_hbm, out_hbm, idx_v, rows_v, sem):
        wid = lax.axis_index("s") * num_cores + lax.axis_index("c")
        first_chunk = wid * chunks_per_w
        pltpu.sync_copy(idx_hbm.at[wid], idx_v)

        def body(j, carry):
            cp0 = pltpu.async_copy(table_hbm.at[idx_v.at[j, 0]], rows_v.at[0], sem)
            cp1 = pltpu.async_copy(table_hbm.at[idx_v.at[j, 1]], rows_v.at[1], sem)
            cp0.wait()
            cp1.wait()
            off = pl.multiple_of((first_chunk + j) * chunk, chunk)
            pltpu.sync_copy(rows_v.at[0], out_hbm.at[pl.ds(off, chunk), pl.ds(0, depth)])
            pltpu.sync_copy(rows_v.at[1], out_hbm.at[pl.ds(off, chunk), pl.ds(depth, depth)])
            return carry

        lax.fori_loop(0, chunks_per_w, body, 0)

    return gather_kernel(table, ids4)


def _tc_project_norm(gathered, W2, rms_weight2, d_model, block_rows):
    total_rows, width = gathered.shape
    out_width = 2 * d_model
    grid = total_rows // block_rows

    def body(g_ref, w_ref, rw_ref, o_ref):
        y = lax.dot_general(
            g_ref[...], w_ref[...], (((1,), (0,)), ((), ())),
            preferred_element_type=jnp.float32,
        )
        y_lo = y[:, :d_model]
        y_hi = y[:, d_model:]
        s_lo = lax.rsqrt(jnp.mean(y_lo * y_lo, axis=-1, keepdims=True) + 1e-8)
        s_hi = lax.rsqrt(jnp.mean(y_hi * y_hi, axis=-1, keepdims=True) + 1e-8)
        rw = rw_ref[...]
        o_ref[:, :d_model] = y_lo * s_lo * rw[:, :d_model]
        o_ref[:, d_model:] = y_hi * s_hi * rw[:, d_model:]

    return pl.pallas_call(
        body,
        grid=(grid,),
        in_specs=[
            pl.BlockSpec((block_rows, width), lambda i: (i, 0)),
            pl.BlockSpec((width, out_width), lambda i: (0, 0)),
            pl.BlockSpec((1, out_width), lambda i: (0, 0)),
        ],
        out_specs=pl.BlockSpec((block_rows, out_width), lambda i: (i, 0)),
        out_shape=jax.ShapeDtypeStruct((total_rows, out_width), jnp.float32),
        compiler_params=pltpu.CompilerParams(
            dimension_semantics=("arbitrary",),
        ),
    )(gathered, W2, rms_weight2)


def kernel(token_ids, basin_coords, W, rms_weight):
    batch, seq = token_ids.shape
    d_model, depth = W.shape
    info = plsc.get_sparse_core_info()
    num_workers = info.num_cores * info.num_subcores

    ids = token_ids.reshape(-1).astype(jnp.int32)
    ids4 = ids.reshape(num_workers, -1, _GCHUNK, 2).transpose(0, 1, 3, 2)
    gathered = _sc_gather(basin_coords, ids4)

    Wt = W.T
    z = jnp.zeros_like(Wt)
    W2 = jnp.concatenate(
        [jnp.concatenate([Wt, z], axis=1), jnp.concatenate([z, Wt], axis=1)],
        axis=0,
    )
    rw2 = jnp.concatenate([rms_weight, rms_weight]).reshape(1, 2 * d_model)

    out = _tc_project_norm(gathered, W2, rw2, d_model, 1024)
    return out.reshape(batch, seq, d_model)


# speedup vs baseline: 1.6333x; 1.6333x over previous
import functools

import jax
import jax.numpy as jnp
from jax import lax
from jax.experimental import pallas as pl
from jax.experimental.pallas import tpu as pltpu
from jax.experimental.pallas import tpu_sc as plsc

_GCHUNK = 128


def _sc_gather(table, ids3):
    num_workers, chunks_per_w, chunk = ids3.shape
    depth = table.shape[1]
    out_rows = num_workers * chunks_per_w * chunk

    info = plsc.get_sparse_core_info()
    num_cores = info.num_cores

    mesh = plsc.VectorSubcoreMesh(core_axis_name="c", subcore_axis_name="s")

    @functools.partial(
        pl.kernel,
        mesh=mesh,
        out_type=jax.ShapeDtypeStruct((out_rows, 2 * depth), jnp.float32),
        scratch_types=[
            pltpu.VMEM((chunks_per_w, chunk), jnp.int32),
            pltpu.VMEM((chunk, depth), jnp.float32),
            pltpu.VMEM((chunk, depth), jnp.float32),
            pltpu.SemaphoreType.DMA,
        ],
        compiler_params=pltpu.CompilerParams(use_tc_tiling_on_sc=False),
    )
    def gather_kernel(table_hbm, idx_hbm, out_hbm, idx_v, rows_v, zero_v, sem):
        wid = lax.axis_index("s") * num_cores + lax.axis_index("c")
        first_chunk = wid * chunks_per_w
        pltpu.sync_copy(idx_hbm.at[wid], idx_v)

        zeros16 = jnp.zeros((16,), jnp.float32)

        def zbody(i, carry):
            for j in range(depth // 16):
                zero_v[i, pl.ds(j * 16, 16)] = zeros16
            return carry

        lax.fori_loop(0, chunk, zbody, 0)

        def body(j, carry):
            pltpu.async_copy(table_hbm.at[idx_v.at[j]], rows_v, sem).wait()
            off = pl.multiple_of((first_chunk + j) * chunk, chunk)
            pltpu.sync_copy(rows_v, out_hbm.at[pl.ds(off, chunk), pl.ds(0, depth)])
            pltpu.sync_copy(zero_v, out_hbm.at[pl.ds(off, chunk), pl.ds(depth, depth)])
            return carry

        lax.fori_loop(0, chunks_per_w, body, 0)

    return gather_kernel(table, ids3)


def _tc_project_norm(gathered, W2, rms_weight, d_model, block_rows):
    total_rows, width = gathered.shape
    grid = total_rows // block_rows

    def body(g_ref, w_ref, rw_ref, o_ref):
        y = lax.dot_general(
            g_ref[...], w_ref[...], (((1,), (0,)), ((), ())),
            preferred_element_type=jnp.float32,
        )
        ms = jnp.mean(y * y, axis=-1, keepdims=True)
        o_ref[...] = y * lax.rsqrt(ms + 1e-8) * rw_ref[...]

    return pl.pallas_call(
        body,
        grid=(grid,),
        in_specs=[
            pl.BlockSpec((block_rows, width), lambda i: (i, 0)),
            pl.BlockSpec((width, d_model), lambda i: (0, 0)),
            pl.BlockSpec((1, d_model), lambda i: (0, 0)),
        ],
        out_specs=pl.BlockSpec((block_rows, d_model), lambda i: (i, 0)),
        out_shape=jax.ShapeDtypeStruct((total_rows, d_model), jnp.float32),
        compiler_params=pltpu.CompilerParams(
            dimension_semantics=("arbitrary",),
        ),
    )(gathered, W2, rms_weight)


def kernel(token_ids, basin_coords, W, rms_weight):
    batch, seq = token_ids.shape
    d_model, depth = W.shape
    info = plsc.get_sparse_core_info()
    num_workers = info.num_cores * info.num_subcores

    ids = token_ids.reshape(-1).astype(jnp.int32)
    ids3 = ids.reshape(num_workers, -1, _GCHUNK)
    gathered = _sc_gather(basin_coords, ids3)

    W2 = jnp.concatenate([W.T, jnp.zeros_like(W.T)], axis=0)
    rw2 = rms_weight.reshape(1, d_model)

    out = _tc_project_norm(gathered, W2, rw2, d_model, 2048)
    return out.reshape(batch, seq, d_model)


# speedup vs baseline: 1.6621x; 1.0176x over previous
import functools

import jax
import jax.numpy as jnp
from jax import lax
from jax.experimental import pallas as pl
from jax.experimental.pallas import tpu as pltpu
from jax.experimental.pallas import tpu_sc as plsc

_GCHUNK = 128


def _sc_gather(table, ids3):
    num_workers, chunks_per_w, chunk = ids3.shape
    depth = table.shape[1]
    out_rows = num_workers * chunks_per_w * chunk

    info = plsc.get_sparse_core_info()
    num_cores = info.num_cores

    mesh = plsc.VectorSubcoreMesh(core_axis_name="c", subcore_axis_name="s")

    @functools.partial(
        pl.kernel,
        mesh=mesh,
        out_type=jax.ShapeDtypeStruct((out_rows, 2 * depth), jnp.float32),
        scratch_types=[
            pltpu.VMEM((chunks_per_w, chunk), jnp.int32),
            pltpu.VMEM((chunk, depth), jnp.float32),
            pltpu.VMEM((chunk, depth), jnp.float32),
            pltpu.SemaphoreType.DMA,
        ],
        compiler_params=pltpu.CompilerParams(use_tc_tiling_on_sc=False),
    )
    def gather_kernel(table_hbm, idx_hbm, out_hbm, idx_v, rows_v, zero_v, sem):
        wid = lax.axis_index("s") * num_cores + lax.axis_index("c")
        first_chunk = wid * chunks_per_w
        pltpu.sync_copy(idx_hbm.at[wid], idx_v)

        zeros16 = jnp.zeros((16,), jnp.float32)

        def zbody(i, carry):
            for j in range(depth // 16):
                zero_v[i, pl.ds(j * 16, 16)] = zeros16
            return carry

        lax.fori_loop(0, chunk, zbody, 0)

        def body(j, carry):
            pltpu.async_copy(table_hbm.at[idx_v.at[j]], rows_v, sem).wait()
            off = pl.multiple_of((first_chunk + j) * chunk, chunk)
            pltpu.sync_copy(rows_v, out_hbm.at[pl.ds(off, chunk), pl.ds(0, depth)])
            pltpu.sync_copy(zero_v, out_hbm.at[pl.ds(off, chunk), pl.ds(depth, depth)])
            return carry

        lax.fori_loop(0, chunks_per_w, body, 0)

    return gather_kernel(table, ids3)


def _tc_project_norm(gathered, W2, rms_weight, d_model, block_rows):
    total_rows, width = gathered.shape
    grid = total_rows // block_rows

    def body(g_ref, w_ref, rw_ref, o_ref):
        y = lax.dot_general(
            g_ref[...], w_ref[...], (((1,), (0,)), ((), ())),
            preferred_element_type=jnp.float32,
        )
        ms = jnp.mean(y * y, axis=-1, keepdims=True)
        o_ref[...] = y * lax.rsqrt(ms + 1e-8) * rw_ref[...]

    return pl.pallas_call(
        body,
        grid=(grid,),
        in_specs=[
            pl.BlockSpec((block_rows, width), lambda i: (i, 0)),
            pl.BlockSpec((width, d_model), lambda i: (0, 0)),
            pl.BlockSpec((1, d_model), lambda i: (0, 0)),
        ],
        out_specs=pl.BlockSpec((block_rows, d_model), lambda i: (i, 0)),
        out_shape=jax.ShapeDtypeStruct((total_rows, d_model), jnp.float32),
        compiler_params=pltpu.CompilerParams(
            dimension_semantics=("arbitrary",),
        ),
    )(gathered, W2, rms_weight)


def kernel(token_ids, basin_coords, W, rms_weight):
    batch, seq = token_ids.shape
    d_model, depth = W.shape
    info = plsc.get_sparse_core_info()
    num_workers = info.num_cores * info.num_subcores

    ids = token_ids.reshape(-1).astype(jnp.int32)
    ids3 = ids.reshape(num_workers, -1, _GCHUNK)
    gathered = _sc_gather(basin_coords, ids3)

    W2 = jnp.concatenate([W.T, jnp.zeros_like(W.T)], axis=0)
    rw2 = rms_weight.reshape(1, d_model)

    out = _tc_project_norm(gathered, W2, rw2, d_model, 4096)
    return out.reshape(batch, seq, d_model)


# speedup vs baseline: 1.7181x; 1.0337x over previous
import functools

import jax
import jax.numpy as jnp
from jax import lax
from jax.experimental import pallas as pl
from jax.experimental.pallas import tpu as pltpu
from jax.experimental.pallas import tpu_sc as plsc

_GCHUNK = 128


def _sc_gather(table, ids3):
    num_workers, chunks_per_w, chunk = ids3.shape
    depth = table.shape[1]
    out_rows = num_workers * chunks_per_w * chunk

    info = plsc.get_sparse_core_info()
    num_cores = info.num_cores

    mesh = plsc.VectorSubcoreMesh(core_axis_name="c", subcore_axis_name="s")

    @functools.partial(
        pl.kernel,
        mesh=mesh,
        out_type=jax.ShapeDtypeStruct((out_rows, 2 * depth), jnp.float32),
        scratch_types=[
            pltpu.VMEM((chunks_per_w, chunk), jnp.int32),
            pltpu.VMEM((2, chunk, depth), jnp.float32),
            pltpu.VMEM((chunk, depth), jnp.float32),
            pltpu.SemaphoreType.DMA,
            pltpu.SemaphoreType.DMA,
            pltpu.SemaphoreType.DMA,
        ],
        compiler_params=pltpu.CompilerParams(use_tc_tiling_on_sc=False),
    )
    def gather_kernel(table_hbm, idx_hbm, out_hbm, idx_v, rows_v, zero_v,
                      gsem0, gsem1, ssem):
        wid = lax.axis_index("s") * num_cores + lax.axis_index("c")
        first_chunk = wid * chunks_per_w
        pltpu.sync_copy(idx_hbm.at[wid], idx_v)

        zeros16 = jnp.zeros((16,), jnp.float32)

        def zbody(i, carry):
            for j in range(depth // 16):
                zero_v[i, pl.ds(j * 16, 16)] = zeros16
            return carry

        lax.fori_loop(0, chunk, zbody, 0)

        def out_rows_ref(j):
            off = pl.multiple_of((first_chunk + j) * chunk, chunk)
            return out_hbm.at[pl.ds(off, chunk), pl.ds(0, depth)]

        def out_zero_ref(j):
            off = pl.multiple_of((first_chunk + j) * chunk, chunk)
            return out_hbm.at[pl.ds(off, chunk), pl.ds(depth, depth)]

        def wait_scatter(slot, j):
            pltpu.make_async_copy(rows_v.at[slot], out_rows_ref(j), ssem).wait()
            pltpu.make_async_copy(zero_v, out_zero_ref(j), ssem).wait()

        pltpu.async_copy(table_hbm.at[idx_v.at[0]], rows_v.at[0], gsem0)

        def body2(j, carry):
            slot = lax.rem(j, 2)
            nslot = 1 - slot

            @pl.when(j + 1 < chunks_per_w)
            def _():
                @pl.when(j >= 1)
                def _():
                    wait_scatter(nslot, j - 1)

                @pl.when(nslot == 0)
                def _():
                    pltpu.async_copy(
                        table_hbm.at[idx_v.at[j + 1]], rows_v.at[0], gsem0)

                @pl.when(nslot == 1)
                def _():
                    pltpu.async_copy(
                        table_hbm.at[idx_v.at[j + 1]], rows_v.at[1], gsem1)

            @pl.when(slot == 0)
            def _():
                pltpu.make_async_copy(
                    table_hbm.at[idx_v.at[0]], rows_v.at[0], gsem0).wait()

            @pl.when(slot == 1)
            def _():
                pltpu.make_async_copy(
                    table_hbm.at[idx_v.at[0]], rows_v.at[1], gsem1).wait()

            pltpu.async_copy(rows_v.at[slot], out_rows_ref(j), ssem)
            pltpu.async_copy(zero_v, out_zero_ref(j), ssem)
            return carry

        lax.fori_loop(0, chunks_per_w, body2, 0)
        wait_scatter(lax.rem(chunks_per_w - 1, 2), chunks_per_w - 1)
        wait_scatter(lax.rem(chunks_per_w, 2), chunks_per_w - 2)

    return gather_kernel(table, ids3)


def _tc_project_norm(gathered, W2, rms_weight, d_model, block_rows):
    total_rows, width = gathered.shape
    grid = total_rows // block_rows

    def body(g_ref, w_ref, rw_ref, o_ref):
        y = lax.dot_general(
            g_ref[...], w_ref[...], (((1,), (0,)), ((), ())),
            preferred_element_type=jnp.float32,
        )
        ms = jnp.mean(y * y, axis=-1, keepdims=True)
        o_ref[...] = y * lax.rsqrt(ms + 1e-8) * rw_ref[...]

    return pl.pallas_call(
        body,
        grid=(grid,),
        in_specs=[
            pl.BlockSpec((block_rows, width), lambda i: (i, 0)),
            pl.BlockSpec((width, d_model), lambda i: (0, 0)),
            pl.BlockSpec((1, d_model), lambda i: (0, 0)),
        ],
        out_specs=pl.BlockSpec((block_rows, d_model), lambda i: (i, 0)),
        out_shape=jax.ShapeDtypeStruct((total_rows, d_model), jnp.float32),
        compiler_params=pltpu.CompilerParams(
            dimension_semantics=("arbitrary",),
        ),
    )(gathered, W2, rms_weight)


def kernel(token_ids, basin_coords, W, rms_weight):
    batch, seq = token_ids.shape
    d_model, depth = W.shape
    info = plsc.get_sparse_core_info()
    num_workers = info.num_cores * info.num_subcores

    ids = token_ids.reshape(-1).astype(jnp.int32)
    ids3 = ids.reshape(num_workers, -1, _GCHUNK)
    gathered = _sc_gather(basin_coords, ids3)

    W2 = jnp.concatenate([W.T, jnp.zeros_like(W.T)], axis=0)
    rw2 = rms_weight.reshape(1, d_model)

    out = _tc_project_norm(gathered, W2, rw2, d_model, 4096)
    return out.reshape(batch, seq, d_model)
